# Initial kernel scaffold; baseline (speedup 1.0000x reference)
import jax
import jax.numpy as jnp
from jax.experimental import pallas as pl


def kernel(x, edge_index, batch, W1, b1, W2, b2, fc1_w, fc1_b, fc2_w, fc2_b):
    raise NotImplementedError("write your pallas kernel here")


# speedup vs baseline: 35.2686x; 35.2686x over previous
import functools

import jax
import jax.numpy as jnp
from jax import lax
from jax.experimental import pallas as pl
from jax.experimental.pallas import tpu as pltpu
from jax.experimental.pallas import tpu_sc as plsc

N = 10000
E = 320000
G = 32
DIN = 128
H = 64
NCLS = 10

NC, NS = 2, 16
NW = NC * NS
K = 80
CH = (E // NW) // K
EROWS = E // K
NPAD = 10240
DEG_PER_SUB = NPAD // NS
ROWS_PER_SUB = N // NS

_f32 = jnp.float32
_mesh = plsc.VectorSubcoreMesh(core_axis_name="c", subcore_axis_name="s")
_sc_params = pltpu.CompilerParams(use_tc_tiling_on_sc=False)


def _deg_body(dst_hbm, out_hbm, didx_v, ones_v, zb_v, deg_sh):
    c = lax.axis_index("c")
    s = lax.axis_index("s")
    w = c * NS + s

    def _fill_z(i, _):
        zb_v[pl.ds(i * 16, 16)] = jnp.zeros((16,), _f32)
        return 0

    lax.fori_loop(0, DEG_PER_SUB // 16, _fill_z, 0)

    def _fill_o(i, _):
        ones_v[pl.ds(i * 16, 16)] = jnp.ones((16,), _f32)
        return 0

    lax.fori_loop(0, K // 16, _fill_o, 0)

    pltpu.sync_copy(dst_hbm.at[w], didx_v)
    pltpu.sync_copy(zb_v, deg_sh.at[pl.ds(s * DEG_PER_SUB, DEG_PER_SUB)])
    plsc.subcore_barrier()

    def _scat(i, _):
        pltpu.sync_copy(ones_v, deg_sh.at[didx_v.at[i]], add=True)
        return 0

    lax.fori_loop(0, CH, _scat, 0)
    plsc.subcore_barrier()
    pltpu.sync_copy(deg_sh.at[pl.ds(s * DEG_PER_SUB, DEG_PER_SUB)],
                    out_hbm.at[c, s, 0])


_deg_call = functools.partial(
    pl.kernel,
    out_type=jax.ShapeDtypeStruct((NC, NS, 1, DEG_PER_SUB), _f32),
    mesh=_mesh,
    compiler_params=_sc_params,
    scratch_types=[
        pltpu.VMEM((CH, K), jnp.int32),
        pltpu.VMEM((K,), _f32),
        pltpu.VMEM((DEG_PER_SUB,), _f32),
        pltpu.VMEM_SHARED((NPAD,), _f32),
    ],
)(_deg_body)


def _edge_body(g_hbm, sidx_hbm, didx_hbm, out_hbm,
               sidx_v, didx_v, rows0, rows1, zb_v, acc_sh, gsem0, gsem1):
    c = lax.axis_index("c")
    s = lax.axis_index("s")
    w = c * NS + s

    pltpu.sync_copy(sidx_hbm.at[w], sidx_v)
    pltpu.sync_copy(didx_hbm.at[w], didx_v)

    def _fill_z(i, _):
        zb_v[i // (H // 16), pl.ds((i % (H // 16)) * 16, 16)] = jnp.zeros((16,), _f32)
        return 0

    lax.fori_loop(0, 125 * (H // 16), _fill_z, 0)

    def _zero(j, _):
        pltpu.sync_copy(zb_v, acc_sh.at[pl.ds(s * ROWS_PER_SUB + j * 125, 125)])
        return 0

    lax.fori_loop(0, ROWS_PER_SUB // 125, _zero, 0)
    plsc.subcore_barrier()

    pltpu.async_copy(g_hbm.at[sidx_v.at[0]], rows0, gsem0)

    def _body(k, _):
        i = k * 2
        pltpu.async_copy(g_hbm.at[sidx_v.at[i + 1]], rows1, gsem1)
        pltpu.make_async_copy(g_hbm.at[sidx_v.at[i]], rows0, gsem0).wait()
        pltpu.sync_copy(rows0, acc_sh.at[didx_v.at[i]], add=True)
        pltpu.async_copy(g_hbm.at[sidx_v.at[i + 2]], rows0, gsem0)
        pltpu.make_async_copy(g_hbm.at[sidx_v.at[i + 1]], rows1, gsem1).wait()
        pltpu.sync_copy(rows1, acc_sh.at[didx_v.at[i + 1]], add=True)
        return 0

    lax.fori_loop(0, (CH - 1) // 2, _body, 0)
    pltpu.make_async_copy(g_hbm.at[sidx_v.at[CH - 1]], rows0, gsem0).wait()
    pltpu.sync_copy(rows0, acc_sh.at[didx_v.at[CH - 1]], add=True)

    plsc.subcore_barrier()
    pltpu.sync_copy(acc_sh.at[pl.ds(s * ROWS_PER_SUB, ROWS_PER_SUB)],
                    out_hbm.at[c, s])


_edge_call = functools.partial(
    pl.kernel,
    out_type=jax.ShapeDtypeStruct((NC, NS, ROWS_PER_SUB, H), _f32),
    mesh=_mesh,
    compiler_params=_sc_params,
    scratch_types=[
        pltpu.VMEM((CH, K), jnp.int32),
        pltpu.VMEM((CH, K), jnp.int32),
        pltpu.VMEM((K, H), _f32),
        pltpu.VMEM((K, H), _f32),
        pltpu.VMEM((125, H), _f32),
        pltpu.VMEM_SHARED((N, H), _f32),
        pltpu.SemaphoreType.DMA,
        pltpu.SemaphoreType.DMA,
    ],
)(_edge_body)


def _tca_body(degp_ref, x_ref, w1_ref, dinv_ref, g1_ref):
    deg = degp_ref[0] + degp_ref[1] + 1.0
    dinv = lax.rsqrt(jnp.maximum(deg, 1.0))
    dinv_ref[...] = dinv
    p1 = jnp.dot(x_ref[...], w1_ref[...], preferred_element_type=_f32)
    g1_ref[...] = dinv[:N] * p1


def _tca_call(degp, x, w1):
    return pl.pallas_call(
        _tca_body,
        out_shape=[jax.ShapeDtypeStruct((NPAD, 1), _f32),
                   jax.ShapeDtypeStruct((N, H), _f32)],
    )(degp, x, w1)


def _tcb_body(acc_ref, g1_ref, dinv_ref, b1_ref, g1b_ref):
    dinv = dinv_ref[...][:N]
    h = jax.nn.relu(dinv * (acc_ref[0] + acc_ref[1] + g1_ref[...]) + b1_ref[...])
    g1b_ref[...] = dinv * h


def _tcb_call(acc, g1, dinv, b1):
    return pl.pallas_call(
        _tcb_body,
        out_shape=jax.ShapeDtypeStruct((N, H), _f32),
    )(acc, g1, dinv, b1)


def _tcc_body(acc_ref, g1b_ref, dinv_ref, w2_ref, b2_ref, batch_ref,
              fc1w_ref, fc1b_ref, fc2w_ref, fc2b_ref, out_ref):
    t = jnp.dot(acc_ref[0] + acc_ref[1] + g1b_ref[...], w2_ref[...],
                preferred_element_type=_f32)
    h2 = jax.nn.relu(dinv_ref[...][:N] * t + b2_ref[...])
    onehot = (batch_ref[...] ==
              lax.broadcasted_iota(jnp.int32, (N, G), 1)).astype(_f32)
    sums = lax.dot_general(onehot, h2, (((0,), (0,)), ((), ())),
                           preferred_element_type=_f32)
    counts = lax.dot_general(onehot, jnp.ones((N, 1), _f32),
                             (((0,), (0,)), ((), ())),
                             preferred_element_type=_f32)
    pooled = sums / jnp.maximum(counts, 1.0)
    z = jax.nn.relu(jnp.dot(pooled, fc1w_ref[...],
                            preferred_element_type=_f32) + fc1b_ref[...])
    out_ref[...] = jnp.dot(z, fc2w_ref[...],
                           preferred_element_type=_f32) + fc2b_ref[...]


def _tcc_call(acc, g1b, dinv, w2, b2, batch, fc1w, fc1b, fc2w, fc2b):
    return pl.pallas_call(
        _tcc_body,
        out_shape=jax.ShapeDtypeStruct((G, NCLS), _f32),
    )(acc, g1b, dinv, w2, b2, batch, fc1w, fc1b, fc2w, fc2b)


def kernel(x, edge_index, batch, W1, b1, W2, b2, fc1_w, fc1_b, fc2_w, fc2_b):
    ei = edge_index.astype(jnp.int32)
    src3d = ei[0].reshape(NW, CH, K)
    dst3d = ei[1].reshape(NW, CH, K)

    degp = _deg_call(dst3d)
    dinv, g1 = _tca_call(degp.reshape(NC, NPAD, 1), x, W1)
    s1 = _edge_call(g1, src3d, dst3d).reshape(NC, N, H)
    g1b = _tcb_call(s1, g1, dinv, b1.reshape(1, H))
    s2 = _edge_call(g1b, src3d, dst3d).reshape(NC, N, H)
    out = _tcc_call(s2, g1b, dinv, W2, b2.reshape(1, 2 * H),
                    batch.reshape(N, 1), fc1_w, fc1_b.reshape(1, H),
                    fc2_w, fc2_b.reshape(1, NCLS))
    return out


# speedup vs baseline: 40.4951x; 1.1482x over previous
import functools

import jax
import jax.numpy as jnp
from jax import lax
from jax.experimental import pallas as pl
from jax.experimental.pallas import tpu as pltpu
from jax.experimental.pallas import tpu_sc as plsc

N = 10000
E = 320000
G = 32
DIN = 128
H = 64
NCLS = 10

NC, NS = 2, 16
NW = NC * NS
K = 80
CH = (E // NW) // K
EROWS = E // K
NPAD = 10240
DEG_PER_SUB = NPAD // NS
ROWS_PER_SUB = N // NS

_f32 = jnp.float32
_mesh = plsc.VectorSubcoreMesh(core_axis_name="c", subcore_axis_name="s")
_sc_params = pltpu.CompilerParams(use_tc_tiling_on_sc=False)


def _deg_body(dst_hbm, out_hbm, didx_v, ones_v, zb_v, deg_sh):
    c = lax.axis_index("c")
    s = lax.axis_index("s")
    w = c * NS + s

    def _fill_z(i, _):
        zb_v[pl.ds(i * 16, 16)] = jnp.zeros((16,), _f32)
        return 0

    lax.fori_loop(0, DEG_PER_SUB // 16, _fill_z, 0)

    def _fill_o(i, _):
        ones_v[pl.ds(i * 16, 16)] = jnp.ones((16,), _f32)
        return 0

    lax.fori_loop(0, K // 16, _fill_o, 0)

    pltpu.sync_copy(dst_hbm.at[w], didx_v)
    pltpu.sync_copy(zb_v, deg_sh.at[pl.ds(s * DEG_PER_SUB, DEG_PER_SUB)])
    plsc.subcore_barrier()

    def _scat(i, _):
        pltpu.sync_copy(ones_v, deg_sh.at[didx_v.at[i]], add=True)
        return 0

    lax.fori_loop(0, CH, _scat, 0)
    plsc.subcore_barrier()
    pltpu.sync_copy(deg_sh.at[pl.ds(s * DEG_PER_SUB, DEG_PER_SUB)],
                    out_hbm.at[c, s, 0])


_deg_call = functools.partial(
    pl.kernel,
    out_type=jax.ShapeDtypeStruct((NC, NS, 1, DEG_PER_SUB), _f32),
    mesh=_mesh,
    compiler_params=_sc_params,
    scratch_types=[
        pltpu.VMEM((CH, K), jnp.int32),
        pltpu.VMEM((K,), _f32),
        pltpu.VMEM((DEG_PER_SUB,), _f32),
        pltpu.VMEM_SHARED((NPAD,), _f32),
    ],
)(_deg_body)


NBUF = 4


def _edge_body(g_hbm, sidx_hbm, didx_hbm, out_hbm,
               sidx_v, didx_v, rows, zb_v, acc_sh, gsems, ssems):
    c = lax.axis_index("c")
    s = lax.axis_index("s")
    w = c * NS + s

    pltpu.sync_copy(sidx_hbm.at[w], sidx_v)
    pltpu.sync_copy(didx_hbm.at[w], didx_v)

    def _fill_z(i, _):
        zb_v[i // (H // 16), pl.ds((i % (H // 16)) * 16, 16)] = jnp.zeros((16,), _f32)
        return 0

    lax.fori_loop(0, 125 * (H // 16), _fill_z, 0)

    def _zero(j, _):
        pltpu.sync_copy(zb_v, acc_sh.at[pl.ds(s * ROWS_PER_SUB + j * 125, 125)])
        return 0

    lax.fori_loop(0, ROWS_PER_SUB // 125, _zero, 0)
    plsc.subcore_barrier()

    def _gather(i, b):
        pltpu.async_copy(g_hbm.at[sidx_v.at[i]], rows[b], gsems[b])

    def _gwait(i, b):
        pltpu.make_async_copy(g_hbm.at[sidx_v.at[i]], rows[b], gsems[b]).wait()

    def _scat(i, b):
        pltpu.async_copy(rows[b], acc_sh.at[didx_v.at[i]], ssems[b], add=True)

    def _swait(i, b):
        pltpu.make_async_copy(rows[b], acc_sh.at[didx_v.at[i]], ssems[b]).wait()

    for b in range(NBUF):
        _gather(b, b)

    def _body(k, _):
        i0 = k * NBUF
        for b in range(NBUF):
            _gwait(i0 + b, b)
            _scat(i0 + b, b)
        for b in range(NBUF):
            _swait(i0 + b, b)

            @pl.when(i0 + NBUF + b < CH)
            def _():
                _gather(i0 + NBUF + b, b)
        return 0

    nfull = CH // NBUF
    lax.fori_loop(0, nfull, _body, 0)
    for i in range(nfull * NBUF, CH):
        b = i % NBUF
        _gwait(i, b)
        _scat(i, b)
        _swait(i, b)

    plsc.subcore_barrier()
    pltpu.sync_copy(acc_sh.at[pl.ds(s * ROWS_PER_SUB, ROWS_PER_SUB)],
                    out_hbm.at[c, s])


_edge_call = functools.partial(
    pl.kernel,
    out_type=jax.ShapeDtypeStruct((NC, NS, ROWS_PER_SUB, H), _f32),
    mesh=_mesh,
    compiler_params=_sc_params,
    scratch_types=[
        pltpu.VMEM((CH, K), jnp.int32),
        pltpu.VMEM((CH, K), jnp.int32),
        [pltpu.VMEM((K, H), _f32)] * NBUF,
        pltpu.VMEM((125, H), _f32),
        pltpu.VMEM_SHARED((N, H), _f32),
        [pltpu.SemaphoreType.DMA] * NBUF,
        [pltpu.SemaphoreType.DMA] * NBUF,
    ],
)(_edge_body)


def _tca_body(degp_ref, x_ref, w1_ref, dinv_ref, g1_ref):
    deg = degp_ref[0] + degp_ref[1] + 1.0
    dinv = lax.rsqrt(jnp.maximum(deg, 1.0))
    dinv_ref[...] = dinv
    p1 = jnp.dot(x_ref[...], w1_ref[...], preferred_element_type=_f32)
    g1_ref[...] = dinv[:N] * p1


def _tca_call(degp, x, w1):
    return pl.pallas_call(
        _tca_body,
        out_shape=[jax.ShapeDtypeStruct((NPAD, 1), _f32),
                   jax.ShapeDtypeStruct((N, H), _f32)],
    )(degp, x, w1)


def _tcb_body(acc_ref, g1_ref, dinv_ref, b1_ref, g1b_ref):
    dinv = dinv_ref[...][:N]
    h = jax.nn.relu(dinv * (acc_ref[0] + acc_ref[1] + g1_ref[...]) + b1_ref[...])
    g1b_ref[...] = dinv * h


def _tcb_call(acc, g1, dinv, b1):
    return pl.pallas_call(
        _tcb_body,
        out_shape=jax.ShapeDtypeStruct((N, H), _f32),
    )(acc, g1, dinv, b1)


def _tcc_body(acc_ref, g1b_ref, dinv_ref, w2_ref, b2_ref, batch_ref,
              fc1w_ref, fc1b_ref, fc2w_ref, fc2b_ref, out_ref):
    t = jnp.dot(acc_ref[0] + acc_ref[1] + g1b_ref[...], w2_ref[...],
                preferred_element_type=_f32)
    h2 = jax.nn.relu(dinv_ref[...][:N] * t + b2_ref[...])
    onehot = (batch_ref[...] ==
              lax.broadcasted_iota(jnp.int32, (N, G), 1)).astype(_f32)
    sums = lax.dot_general(onehot, h2, (((0,), (0,)), ((), ())),
                           preferred_element_type=_f32)
    counts = lax.dot_general(onehot, jnp.ones((N, 1), _f32),
                             (((0,), (0,)), ((), ())),
                             preferred_element_type=_f32)
    pooled = sums / jnp.maximum(counts, 1.0)
    z = jax.nn.relu(jnp.dot(pooled, fc1w_ref[...],
                            preferred_element_type=_f32) + fc1b_ref[...])
    out_ref[...] = jnp.dot(z, fc2w_ref[...],
                           preferred_element_type=_f32) + fc2b_ref[...]


def _tcc_call(acc, g1b, dinv, w2, b2, batch, fc1w, fc1b, fc2w, fc2b):
    return pl.pallas_call(
        _tcc_body,
        out_shape=jax.ShapeDtypeStruct((G, NCLS), _f32),
    )(acc, g1b, dinv, w2, b2, batch, fc1w, fc1b, fc2w, fc2b)


def kernel(x, edge_index, batch, W1, b1, W2, b2, fc1_w, fc1_b, fc2_w, fc2_b):
    ei = edge_index.astype(jnp.int32)
    src3d = ei[0].reshape(NW, CH, K)
    dst3d = ei[1].reshape(NW, CH, K)

    degp = _deg_call(dst3d)
    dinv, g1 = _tca_call(degp.reshape(NC, NPAD, 1), x, W1)
    s1 = _edge_call(g1, src3d, dst3d).reshape(NC, N, H)
    g1b = _tcb_call(s1, g1, dinv, b1.reshape(1, H))
    s2 = _edge_call(g1b, src3d, dst3d).reshape(NC, N, H)
    out = _tcc_call(s2, g1b, dinv, W2, b2.reshape(1, 2 * H),
                    batch.reshape(N, 1), fc1_w, fc1_b.reshape(1, H),
                    fc2_w, fc2_b.reshape(1, NCLS))
    return out


# speedup vs baseline: 42.3177x; 1.0450x over previous
import functools

import jax
import jax.numpy as jnp
from jax import lax
from jax.experimental import pallas as pl
from jax.experimental.pallas import tpu as pltpu
from jax.experimental.pallas import tpu_sc as plsc

N = 10000
E = 320000
G = 32
DIN = 128
H = 64
NCLS = 10

NC, NS = 2, 16
NW = NC * NS
K = 80
CH = (E // NW) // K
EROWS = E // K
NPAD = 10240
DEG_PER_SUB = NPAD // NS
ROWS_PER_SUB = N // NS

_f32 = jnp.float32
_mesh = plsc.VectorSubcoreMesh(core_axis_name="c", subcore_axis_name="s")
_sc_params = pltpu.CompilerParams(use_tc_tiling_on_sc=False)


def _deg_body(dst_hbm, out_hbm, didx_v, ones_v, zb_v, deg_sh):
    c = lax.axis_index("c")
    s = lax.axis_index("s")
    w = c * NS + s

    def _fill_z(i, _):
        zb_v[pl.ds(i * 16, 16)] = jnp.zeros((16,), _f32)
        return 0

    lax.fori_loop(0, DEG_PER_SUB // 16, _fill_z, 0)

    def _fill_o(i, _):
        ones_v[pl.ds(i * 16, 16)] = jnp.ones((16,), _f32)
        return 0

    lax.fori_loop(0, K // 16, _fill_o, 0)

    pltpu.sync_copy(dst_hbm.at[w], didx_v)
    pltpu.sync_copy(zb_v, deg_sh.at[pl.ds(s * DEG_PER_SUB, DEG_PER_SUB)])
    plsc.subcore_barrier()

    def _scat(i, _):
        pltpu.sync_copy(ones_v, deg_sh.at[didx_v.at[i]], add=True)
        return 0

    lax.fori_loop(0, CH, _scat, 0)
    plsc.subcore_barrier()
    pltpu.sync_copy(deg_sh.at[pl.ds(s * DEG_PER_SUB, DEG_PER_SUB)],
                    out_hbm.at[c, s, 0])


_deg_call = functools.partial(
    pl.kernel,
    out_type=jax.ShapeDtypeStruct((NC, NS, 1, DEG_PER_SUB), _f32),
    mesh=_mesh,
    compiler_params=_sc_params,
    scratch_types=[
        pltpu.VMEM((CH, K), jnp.int32),
        pltpu.VMEM((K,), _f32),
        pltpu.VMEM((DEG_PER_SUB,), _f32),
        pltpu.VMEM_SHARED((NPAD,), _f32),
    ],
)(_deg_body)


NBUF = 8


def _edge_body(g_hbm, sidx_hbm, didx_hbm, out_hbm,
               sidx_v, didx_v, rows, zb_v, acc_sh, gsems, ssems):
    c = lax.axis_index("c")
    s = lax.axis_index("s")
    w = c * NS + s

    pltpu.sync_copy(sidx_hbm.at[w], sidx_v)
    pltpu.sync_copy(didx_hbm.at[w], didx_v)

    def _fill_z(i, _):
        zb_v[i // (H // 16), pl.ds((i % (H // 16)) * 16, 16)] = jnp.zeros((16,), _f32)
        return 0

    lax.fori_loop(0, 125 * (H // 16), _fill_z, 0)

    def _zero(j, _):
        pltpu.sync_copy(zb_v, acc_sh.at[pl.ds(s * ROWS_PER_SUB + j * 125, 125)])
        return 0

    lax.fori_loop(0, ROWS_PER_SUB // 125, _zero, 0)
    plsc.subcore_barrier()

    def _gather(i, b):
        pltpu.async_copy(g_hbm.at[sidx_v.at[i]], rows[b], gsems[b])

    def _gwait(i, b):
        pltpu.make_async_copy(g_hbm.at[sidx_v.at[i]], rows[b], gsems[b]).wait()

    def _scat(i, b):
        pltpu.async_copy(rows[b], acc_sh.at[didx_v.at[i]], ssems[b], add=True)

    def _swait(i, b):
        pltpu.make_async_copy(rows[b], acc_sh.at[didx_v.at[i]], ssems[b]).wait()

    for b in range(NBUF):
        _gather(b, b)

    def _body(k, _):
        i0 = k * NBUF
        for b in range(NBUF):
            _gwait(i0 + b, b)
            _scat(i0 + b, b)
        for b in range(NBUF):
            _swait(i0 + b, b)

            @pl.when(i0 + NBUF + b < CH)
            def _():
                _gather(i0 + NBUF + b, b)
        return 0

    nfull = CH // NBUF
    lax.fori_loop(0, nfull, _body, 0)
    for i in range(nfull * NBUF, CH):
        b = i % NBUF
        _gwait(i, b)
        _scat(i, b)
        _swait(i, b)

    plsc.subcore_barrier()
    pltpu.sync_copy(acc_sh.at[pl.ds(s * ROWS_PER_SUB, ROWS_PER_SUB)],
                    out_hbm.at[c, s])


_edge_call = functools.partial(
    pl.kernel,
    out_type=jax.ShapeDtypeStruct((NC, NS, ROWS_PER_SUB, H), _f32),
    mesh=_mesh,
    compiler_params=_sc_params,
    scratch_types=[
        pltpu.VMEM((CH, K), jnp.int32),
        pltpu.VMEM((CH, K), jnp.int32),
        [pltpu.VMEM((K, H), _f32)] * NBUF,
        pltpu.VMEM((125, H), _f32),
        pltpu.VMEM_SHARED((N, H), _f32),
        [pltpu.SemaphoreType.DMA] * NBUF,
        [pltpu.SemaphoreType.DMA] * NBUF,
    ],
)(_edge_body)


def _tca_body(degp_ref, x_ref, w1_ref, dinv_ref, g1_ref):
    deg = degp_ref[0] + degp_ref[1] + 1.0
    dinv = lax.rsqrt(jnp.maximum(deg, 1.0))
    dinv_ref[...] = dinv
    p1 = jnp.dot(x_ref[...], w1_ref[...], preferred_element_type=_f32)
    g1_ref[...] = dinv[:N] * p1


def _tca_call(degp, x, w1):
    return pl.pallas_call(
        _tca_body,
        out_shape=[jax.ShapeDtypeStruct((NPAD, 1), _f32),
                   jax.ShapeDtypeStruct((N, H), _f32)],
    )(degp, x, w1)


def _tcb_body(acc_ref, g1_ref, dinv_ref, b1_ref, g1b_ref):
    dinv = dinv_ref[...][:N]
    h = jax.nn.relu(dinv * (acc_ref[0] + acc_ref[1] + g1_ref[...]) + b1_ref[...])
    g1b_ref[...] = dinv * h


def _tcb_call(acc, g1, dinv, b1):
    return pl.pallas_call(
        _tcb_body,
        out_shape=jax.ShapeDtypeStruct((N, H), _f32),
    )(acc, g1, dinv, b1)


def _tcc_body(acc_ref, g1b_ref, dinv_ref, w2_ref, b2_ref, batch_ref,
              fc1w_ref, fc1b_ref, fc2w_ref, fc2b_ref, out_ref):
    t = jnp.dot(acc_ref[0] + acc_ref[1] + g1b_ref[...], w2_ref[...],
                preferred_element_type=_f32)
    h2 = jax.nn.relu(dinv_ref[...][:N] * t + b2_ref[...])
    onehot = (batch_ref[...] ==
              lax.broadcasted_iota(jnp.int32, (N, G), 1)).astype(_f32)
    sums = lax.dot_general(onehot, h2, (((0,), (0,)), ((), ())),
                           preferred_element_type=_f32)
    counts = lax.dot_general(onehot, jnp.ones((N, 1), _f32),
                             (((0,), (0,)), ((), ())),
                             preferred_element_type=_f32)
    pooled = sums / jnp.maximum(counts, 1.0)
    z = jax.nn.relu(jnp.dot(pooled, fc1w_ref[...],
                            preferred_element_type=_f32) + fc1b_ref[...])
    out_ref[...] = jnp.dot(z, fc2w_ref[...],
                           preferred_element_type=_f32) + fc2b_ref[...]


def _tcc_call(acc, g1b, dinv, w2, b2, batch, fc1w, fc1b, fc2w, fc2b):
    return pl.pallas_call(
        _tcc_body,
        out_shape=jax.ShapeDtypeStruct((G, NCLS), _f32),
    )(acc, g1b, dinv, w2, b2, batch, fc1w, fc1b, fc2w, fc2b)


def kernel(x, edge_index, batch, W1, b1, W2, b2, fc1_w, fc1_b, fc2_w, fc2_b):
    ei = edge_index.astype(jnp.int32)
    src3d = ei[0].reshape(NW, CH, K)
    dst3d = ei[1].reshape(NW, CH, K)

    degp = _deg_call(dst3d)
    dinv, g1 = _tca_call(degp.reshape(NC, NPAD, 1), x, W1)
    s1 = _edge_call(g1, src3d, dst3d).reshape(NC, N, H)
    g1b = _tcb_call(s1, g1, dinv, b1.reshape(1, H))
    s2 = _edge_call(g1b, src3d, dst3d).reshape(NC, N, H)
    out = _tcc_call(s2, g1b, dinv, W2, b2.reshape(1, 2 * H),
                    batch.reshape(N, 1), fc1_w, fc1_b.reshape(1, H),
                    fc2_w, fc2_b.reshape(1, NCLS))
    return out
